# baseline (speedup 1.0000x reference)
import functools

import jax
import jax.numpy as jnp
from jax import lax
from jax.experimental import pallas as pl
from jax.experimental.pallas import tpu as pltpu
from jax.experimental.pallas import tpu_sc as plsc

EPS = 1e-5
N, E, D, DE = 10000, 160000, 256, 16
CIN = 2 * D + DE
NC, NS, L = 2, 16, 16
NW = NC * NS
NEG = -3.0e38

_mesh = plsc.VectorSubcoreMesh(core_axis_name="c", subcore_axis_name="s")


_BN1 = 1000


def _prep_body(x_ref, we_ref, a_ref, b_ref, s1_ref, s2_ref):
    xb = x_ref[...]
    W = we_ref[...]
    s = jnp.sum(W, axis=0)[None, :] / 528.0
    S1 = jnp.sum(xb, axis=1)[:, None]
    S2 = jnp.sum(xb * xb, axis=1)[:, None]
    P = jnp.dot(xb, W[0:256, :], preferred_element_type=jnp.float32) - S1 * s
    Q = jnp.dot(xb, W[272:528, :], preferred_element_type=jnp.float32) - S1 * s

    def rbf16(v):
        vi = lax.bitcast_convert_type(v, jnp.int32)
        return vi + 0x7FFF + (lax.shift_right_logical(vi, 16) & 1)

    def pack(v):
        hi = rbf16(v[:, 0:128]) & jnp.int32(-65536)
        lo = lax.shift_right_logical(rbf16(v[:, 128:256]), 16)
        return hi | lo

    a_ref[...] = pack(P)
    b_ref[...] = pack(Q)
    s1_ref[...] = S1
    s2_ref[...] = S2


def _prep(x, W_e):
    return pl.pallas_call(
        _prep_body,
        grid=(N // _BN1,),
        in_specs=[
            pl.BlockSpec((_BN1, D), lambda i: (i, 0)),
            pl.BlockSpec((CIN, D), lambda i: (0, 0)),
        ],
        out_specs=[
            pl.BlockSpec((_BN1, 128), lambda i: (i, 0)),
            pl.BlockSpec((_BN1, 128), lambda i: (i, 0)),
            pl.BlockSpec((_BN1, 1), lambda i: (i, 0)),
            pl.BlockSpec((_BN1, 1), lambda i: (i, 0)),
        ],
        out_shape=[
            jax.ShapeDtypeStruct((N, 128), jnp.int32),
            jax.ShapeDtypeStruct((N, 128), jnp.int32),
            jax.ShapeDtypeStruct((N, 1), jnp.float32),
            jax.ShapeDtypeStruct((N, 1), jnp.float32),
        ],
    )(x, W_e)


_KB2 = 200
_EPW = E // NW
_EPWP = _EPW + L


@functools.partial(
    pl.kernel,
    out_type=(
        jax.ShapeDtypeStruct((NW, 1, _EPW), jnp.float32),
        jax.ShapeDtypeStruct((NW, 1, _EPW), jnp.float32),
    ),
    mesh=_mesh,
    scratch_types=[
        pltpu.VMEM((_EPWP,), jnp.int32),
        pltpu.VMEM((_EPWP,), jnp.int32),
        pltpu.VMEM((N,), jnp.float32),
        pltpu.VMEM((N,), jnp.float32),
        pltpu.VMEM((_EPWP,), jnp.float32),
        pltpu.VMEM((_EPWP,), jnp.float32),
    ],
    compiler_params=pltpu.CompilerParams(needs_layout_passes=False),
)
def _stats_k(src_hbm, dst_hbm, s1_hbm, s2_hbm, s1p_hbm, s2p_hbm,
             idxs, idxd, s1b, s2b, s1pb, s2pb):
    wid = lax.axis_index("s") * NC + lax.axis_index("c")
    base0 = wid * _EPW
    pltpu.sync_copy(src_hbm.at[pl.ds(base0, _EPW)], idxs.at[pl.ds(0, _EPW)])
    pltpu.sync_copy(dst_hbm.at[pl.ds(base0, _EPW)], idxd.at[pl.ds(0, _EPW)])
    pltpu.sync_copy(s1_hbm, s1b)
    pltpu.sync_copy(s2_hbm, s2b)

    zero16 = jnp.zeros((L,), jnp.int32)
    idxs[pl.ds(_EPW, L)] = zero16
    idxd[pl.ds(_EPW, L)] = zero16

    def stats(i, c):
        sv = idxs[pl.ds(i * L, L)]
        dv = idxd[pl.ds(i * L, L)]
        s1pb[pl.ds(i * L, L)] = (plsc.load_gather(s1b, [sv])
                                 + plsc.load_gather(s1b, [dv]))
        s2pb[pl.ds(i * L, L)] = (plsc.load_gather(s2b, [sv])
                                 + plsc.load_gather(s2b, [dv]))
        return c

    lax.fori_loop(0, (_EPW + L - 1) // L, stats, 0)
    pltpu.sync_copy(s1pb.at[pl.ds(0, _EPW)], s1p_hbm.at[wid, 0])
    pltpu.sync_copy(s2pb.at[pl.ds(0, _EPW)], s2p_hbm.at[wid, 0])


@functools.partial(
    pl.kernel,
    out_type=jax.ShapeDtypeStruct((E, 128), jnp.int32),
    mesh=_mesh,
    scratch_types=[
        pltpu.VMEM((_EPW,), jnp.int32),
        pltpu.VMEM((_EPW,), jnp.int32),
        pltpu.VMEM((_KB2, 128), jnp.int32),
        pltpu.VMEM((_KB2, 128), jnp.int32),
        pltpu.SemaphoreType.DMA,
        pltpu.SemaphoreType.DMA,
        pltpu.SemaphoreType.DMA,
    ],
    compiler_params=pltpu.CompilerParams(needs_layout_passes=False),
)
def _gather_k(src_hbm, dst_hbm, atab_hbm, btab_hbm,
              g_hbm,
              idxs, idxd, bufa, bufb,
              gsa, gsb, wsb):
    wid = lax.axis_index("s") * NC + lax.axis_index("c")
    base0 = wid * _EPW
    pltpu.sync_copy(src_hbm.at[pl.ds(base0, _EPW)], idxs)
    pltpu.sync_copy(dst_hbm.at[pl.ds(base0, _EPW)], idxd)

    def blk(b, carry):
        base = b * _KB2
        cga = pltpu.async_copy(atab_hbm.at[idxs.at[pl.ds(base, _KB2)]],
                               bufa, gsa)
        cgb = pltpu.async_copy(btab_hbm.at[idxd.at[pl.ds(base, _KB2)]],
                               bufb, gsb)
        cga.wait()
        cgb.wait()

        def add(e, cc):
            for j in range(8):
                va = plsc.bitcast(bufa[e, pl.ds(j * L, L)], jnp.bfloat16)
                vb = plsc.bitcast(bufb[e, pl.ds(j * L, L)], jnp.bfloat16)
                bufb[e, pl.ds(j * L, L)] = plsc.bitcast(va + vb, jnp.int32)
            return cc

        lax.fori_loop(0, _KB2, add, 0)
        pltpu.async_copy(bufb, g_hbm.at[pl.ds(base0 + base, _KB2)],
                         wsb).wait()
        return carry

    lax.fori_loop(0, _EPW // _KB2, blk, 0)


_BE3 = E // NW


def _edge_body(g_ref, ea_ref, s1p_ref, s2p_ref, we_ref, be_ref,
               wa_ref, h_ref, attn_ref, af_ref):
    ea = ea_ref[...]
    t = jnp.sum(ea, axis=1, keepdims=True)
    u = jnp.sum(ea * ea, axis=1, keepdims=True)
    s1p = s1p_ref[0, 0, :][:, None]
    s2p = s2p_ref[0, 0, :][:, None]
    mu = (s1p + t) / 528.0
    var = (s2p + u) / 528.0 - mu * mu
    rsig = lax.rsqrt(var + EPS)
    W = we_ref[...]
    s = jnp.sum(W, axis=0)[None, :] / 528.0
    gw = g_ref[...]
    mhi = jnp.int32(-65536)

    def unpack(w, k):
        bits = w & mhi if k == 0 else lax.shift_left(w, 16)
        return lax.bitcast_convert_type(bits, jnp.float32)

    attn = None
    for k in range(2):
        cs = slice(128 * k, 128 * (k + 1))
        gf = unpack(gw, k)
        R = jnp.dot(ea, W[256:272, cs], preferred_element_type=jnp.float32)
        hp = (gf + R - t * s[:, cs]) * rsig + be_ref[...][None, cs]
        h = jnp.where(hp > 0, hp, 0.01 * hp)
        h_ref[:, cs] = h
        pa = jnp.dot(h, wa_ref[cs, :], preferred_element_type=jnp.float32)
        attn = pa if attn is None else attn + pa
    attn_ref[...] = attn
    af_ref[0, 0, :] = attn[:, 0]


def _edge(g, edge_attr, s1p, s2p, W_e, b_e, W_a):
    return pl.pallas_call(
        _edge_body,
        grid=(E // _BE3,),
        in_specs=[
            pl.BlockSpec((_BE3, 128), lambda i: (i, 0)),
            pl.BlockSpec((_BE3, DE), lambda i: (i, 0)),
            pl.BlockSpec((1, 1, _BE3), lambda i: (i, 0, 0)),
            pl.BlockSpec((1, 1, _BE3), lambda i: (i, 0, 0)),
            pl.BlockSpec((CIN, D), lambda i: (0, 0)),
            pl.BlockSpec((D,), lambda i: (0,)),
            pl.BlockSpec((D, 1), lambda i: (0, 0)),
        ],
        out_specs=[
            pl.BlockSpec((_BE3, D), lambda i: (i, 0)),
            pl.BlockSpec((_BE3, 1), lambda i: (i, 0)),
            pl.BlockSpec((1, 1, _BE3), lambda i: (i, 0, 0)),
        ],
        out_shape=[
            jax.ShapeDtypeStruct((E, D), jnp.float32),
            jax.ShapeDtypeStruct((E, 1), jnp.float32),
            jax.ShapeDtypeStruct((NW, 1, _BE3), jnp.float32),
        ],
    )(g, edge_attr, s1p, s2p, W_e, b_e, W_a)


@functools.partial(
    pl.kernel,
    out_type=jax.ShapeDtypeStruct((NW, N), jnp.float32),
    mesh=_mesh,
    scratch_types=[
        pltpu.VMEM((_EPW,), jnp.int32),
        pltpu.VMEM((_EPW,), jnp.float32),
        pltpu.VMEM((N,), jnp.float32),
    ],
    compiler_params=pltpu.CompilerParams(needs_layout_passes=False),
)
def _segmax_k(dst_hbm, a_hbm, mparts_hbm, dbuf, abuf, mbuf):
    wid = lax.axis_index("s") * NC + lax.axis_index("c")
    base = wid * _EPW
    pltpu.sync_copy(dst_hbm.at[pl.ds(base, _EPW)], dbuf)
    pltpu.sync_copy(a_hbm.at[pl.ds(base, _EPW)], abuf)

    def initb(i, c):
        mbuf[pl.ds(i * L, L)] = jnp.full((L,), NEG, jnp.float32)
        return c

    lax.fori_loop(0, N // L, initb, 0)

    def ebody(i, c):
        dv = dbuf[pl.ds(i * L, L)]
        av = abuf[pl.ds(i * L, L)]

        for _ in range(L):
            cur = plsc.load_gather(mbuf, [dv])
            plsc.store_scatter(mbuf, [dv], av, mask=av > cur)
        return c

    lax.fori_loop(0, _EPW // L, ebody, 0)
    pltpu.sync_copy(mbuf, mparts_hbm.at[wid])


def _mmerge_body(mp_ref, m_ref):
    m_ref[...] = jnp.max(mp_ref[...], axis=0, keepdims=True)


def _mmerge(mparts):
    return pl.pallas_call(
        _mmerge_body,
        grid=(1,),
        in_specs=[pl.BlockSpec((NW, N), lambda i: (0, 0))],
        out_specs=pl.BlockSpec((1, N), lambda i: (0, 0)),
        out_shape=jax.ShapeDtypeStruct((1, N), jnp.float32),
    )(mparts)


_KB6 = 80
_EPC = E // NS
_NB6 = _EPC // _KB6
_NPT = 10240
_RPT = _NPT // NS
_CH = D // NC
_DR = _NPT // 128


@functools.partial(
    pl.kernel,
    out_type=(
        jax.ShapeDtypeStruct((_NPT, D), jnp.float32),
        jax.ShapeDtypeStruct((_DR, 128), jnp.float32),
    ),
    mesh=_mesh,
    scratch_types=[
        pltpu.VMEM((N,), jnp.float32),
        pltpu.VMEM((_DR, 128), jnp.float32),
        pltpu.VMEM((_KB6,), jnp.int32),
        pltpu.VMEM((_KB6,), jnp.int32),
        pltpu.VMEM((_KB6,), jnp.float32),
        pltpu.VMEM((_KB6,), jnp.float32),
        pltpu.VMEM((_KB6,), jnp.float32),
        pltpu.VMEM((_KB6, _CH), jnp.float32),
        pltpu.VMEM((_KB6, _CH), jnp.float32),
        pltpu.VMEM((8, 128), jnp.float32),
        pltpu.VMEM_SHARED((_NPT, _CH), jnp.float32),
        pltpu.SemaphoreType.DMA,
        pltpu.SemaphoreType.DMA,
        pltpu.SemaphoreType.DMA,
        pltpu.SemaphoreType.DMA,
        pltpu.SemaphoreType.DMA,
        pltpu.SemaphoreType.DMA,
    ],
    compiler_params=pltpu.CompilerParams(needs_layout_passes=False),
)
def _soft_k(dst_hbm, a_hbm, h_hbm, m_hbm, numer_hbm, den_hbm,
            mbuf, dpbuf, dbuf0, dbuf1, abuf0, abuf1, exb, hbuf0, hbuf1,
            dob, accum, hsem0, hsem1, ssem0, ssem1, isem0, isem1):
    c = lax.axis_index("c")
    sid = lax.axis_index("s")
    colbase = c * _CH
    db = (dbuf0, dbuf1)
    ab = (abuf0, abuf1)
    hb = (hbuf0, hbuf1)
    hsem = (hsem0, hsem1)
    ssem = (ssem0, ssem1)
    isem = (isem0, isem1)
    pltpu.sync_copy(m_hbm.at[0], mbuf)

    zero16 = jnp.zeros((L,), jnp.float32)

    def zdp(e, cc):
        for j in range(8):
            dpbuf[e, pl.ds(j * L, L)] = zero16
        return cc

    lax.fori_loop(0, _DR, zdp, 0)

    def zh(e, cc):
        for j in range(_CH // L):
            hbuf0[e, pl.ds(j * L, L)] = zero16
        return cc

    lax.fori_loop(0, _KB6, zh, 0)
    r0 = sid * _RPT
    for zi in range(_RPT // _KB6):
        pltpu.sync_copy(hbuf0.at[pl.ds(0, _KB6)],
                        accum.at[pl.ds(r0 + zi * _KB6, _KB6)])
    plsc.subcore_barrier()

    def _hslice(b):
        base = sid * _EPC + b * _KB6
        return h_hbm.at[pl.ds(base, _KB6), pl.ds(colbase, _CH)]

    def idxa(b, p):
        base = sid * _EPC + b * _KB6
        pltpu.async_copy(dst_hbm.at[pl.ds(base, _KB6)], db[p], isem[p])
        pltpu.async_copy(a_hbm.at[pl.ds(base, _KB6)], ab[p], isem[p])

    def wait_idxa(b, p):
        base = sid * _EPC + b * _KB6
        pltpu.make_async_copy(dst_hbm.at[pl.ds(base, _KB6)], db[p],
                              isem[p]).wait()
        pltpu.make_async_copy(a_hbm.at[pl.ds(base, _KB6)], ab[p],
                              isem[p]).wait()

    def chunkloop(p):
        def chunk(i, cc):
            dv = db[p][pl.ds(i * L, L)]
            mv = plsc.load_gather(mbuf, [dv])
            av = ab[p][pl.ds(i * L, L)]
            ex = jnp.exp(av - mv)
            exb[pl.ds(i * L, L)] = ex

            @pl.when(c == 0)
            def _():
                row = lax.shift_right_logical(dv, 7)
                col = lax.bitwise_and(dv, jnp.full((L,), 127, jnp.int32))
                plsc.addupdate_scatter(dpbuf, [row, col], ex)

            return cc

        lax.fori_loop(0, _KB6 // L, chunk, 0)

    def scaleloop(p):
        def scale(i, cc):
            exv = exb[pl.ds(i * L, L)]
            for l in range(L):
                e = i * L + l
                exs = exv[l]
                for j in range(_CH // L):
                    hb[p][e, pl.ds(j * L, L)] = (
                        hb[p][e, pl.ds(j * L, L)] * exs)
            return cc

        lax.fori_loop(0, _KB6 // L, scale, 0)

    def slot(b, p, wait_other, nxt):
        wait_idxa(b, p)
        chunkloop(p)
        if wait_other:
            pltpu.make_async_copy(hb[1 - p], accum.at[db[1 - p]],
                                  ssem[1 - p]).wait()
        if nxt:
            idxa(b + 1, 1 - p)
            pltpu.async_copy(_hslice(b + 1), hb[1 - p], hsem[1 - p])
        pltpu.make_async_copy(_hslice(b), hb[p], hsem[p]).wait()
        scaleloop(p)
        pltpu.async_copy(hb[p], accum.at[db[p]], ssem[p], add=True)

    idxa(0, 0)
    pltpu.async_copy(_hslice(0), hb[0], hsem[0])
    slot(0, 0, False, True)

    def pipepair(b2, carry):
        b = 2 * b2 + 1
        slot(b, 1, True, True)
        slot(b + 1, 0, True, True)
        return carry

    lax.fori_loop(0, (_NB6 - 3) // 2, pipepair, 0)
    slot(_NB6 - 2, 1, True, True)
    slot(_NB6 - 1, 0, True, False)
    pltpu.make_async_copy(hb[0], accum.at[db[0]], ssem[0]).wait()
    plsc.subcore_barrier()
    pltpu.sync_copy(accum.at[pl.ds(r0, _RPT)],
                    numer_hbm.at[pl.ds(r0, _RPT), pl.ds(colbase, _CH)])
    plsc.subcore_barrier()

    @pl.when(c == 0)
    def _():
        pltpu.sync_copy(dpbuf, accum.at[pl.ds(sid * _DR, _DR)])

    plsc.subcore_barrier()

    @pl.when((c == 0) & (sid < _DR // 8))
    def _():
        for half in range(2):
            for tt in range(8):
                t = half * 8 + tt
                pltpu.sync_copy(accum.at[pl.ds(t * _DR + sid * 8, 8)],
                                hbuf0.at[pl.ds(tt * 8, 8)])

            def dmerge(i, cc):
                r = i // 8
                j = i % 8
                if half == 0:
                    acc = jnp.zeros((L,), jnp.float32)
                else:
                    acc = dob[r, pl.ds(j * L, L)]
                for tt in range(8):
                    acc = acc + hbuf0[tt * 8 + r, pl.ds(j * L, L)]
                dob[r, pl.ds(j * L, L)] = acc
                return cc

            lax.fori_loop(0, 64, dmerge, 0)
        pltpu.sync_copy(dob, den_hbm.at[pl.ds(sid * 8, 8)])


_BN7 = 1000


def _final_body(x_ref, nu_ref, den_ref, wn_ref, bn_ref, z_ref):
    den = den_ref[...]
    af = nu_ref[...] / jnp.maximum(den, 1e-30)
    feat = jnp.concatenate([x_ref[...], af], axis=1)
    mu = jnp.mean(feat, axis=1, keepdims=True)
    var = jnp.mean((feat - mu) ** 2, axis=1, keepdims=True)
    fn = (feat - mu) * lax.rsqrt(var + EPS)
    z = jnp.dot(fn, wn_ref[...], preferred_element_type=jnp.float32)
    z_ref[...] = jnp.maximum(z + bn_ref[...][None, :], 0.0)


def _final(x, numer, den, W_n, b_n):
    return pl.pallas_call(
        _final_body,
        grid=(N // _BN7,),
        in_specs=[
            pl.BlockSpec((_BN7, D), lambda i: (i, 0)),
            pl.BlockSpec((_BN7, D), lambda i: (i, 0)),
            pl.BlockSpec((_BN7, 1), lambda i: (i, 0)),
            pl.BlockSpec((2 * D, D), lambda i: (0, 0)),
            pl.BlockSpec((D,), lambda i: (0,)),
        ],
        out_specs=pl.BlockSpec((_BN7, D), lambda i: (i, 0)),
        out_shape=jax.ShapeDtypeStruct((N, D), jnp.float32),
    )(x, numer, den, W_n, b_n)


def kernel(x, edge_index, edge_attr, W_e, b_e, W_a, W_n, b_n):
    ei = edge_index.astype(jnp.int32)
    src = ei[0]
    dst = ei[1]
    atab, btab, s1c, s2c = _prep(x, W_e)
    s1p, s2p = _stats_k(src, dst, s1c[:, 0], s2c[:, 0])
    g = _gather_k(src, dst, atab, btab)
    h, attn, af3 = _edge(g, edge_attr, s1p, s2p, W_e, b_e, W_a)
    a_flat = af3.reshape(E)
    mparts = _segmax_k(dst, a_flat)
    mmerged = _mmerge(mparts)
    numer, den_pad = _soft_k(dst, a_flat, h, mmerged)
    den = den_pad.reshape(_NPT)[:N].reshape(N, 1)
    z = _final(x, numer, den, W_n, b_n)
    return z, h, attn


# speedup vs baseline: 1.0187x; 1.0187x over previous
import functools

import jax
import jax.numpy as jnp
from jax import lax
from jax.experimental import pallas as pl
from jax.experimental.pallas import tpu as pltpu
from jax.experimental.pallas import tpu_sc as plsc

EPS = 1e-5
N, E, D, DE = 10000, 160000, 256, 16
CIN = 2 * D + DE
NC, NS, L = 2, 16, 16
NW = NC * NS
NEG = -3.0e38

_mesh = plsc.VectorSubcoreMesh(core_axis_name="c", subcore_axis_name="s")


_BN1 = 1000


def _prep_body(x_ref, we_ref, a_ref, b_ref, s1_ref, s2_ref):
    xb = x_ref[...]
    W = we_ref[...]
    s = jnp.sum(W, axis=0)[None, :] / 528.0
    S1 = jnp.sum(xb, axis=1)[:, None]
    S2 = jnp.sum(xb * xb, axis=1)[:, None]
    P = jnp.dot(xb, W[0:256, :], preferred_element_type=jnp.float32) - S1 * s
    Q = jnp.dot(xb, W[272:528, :], preferred_element_type=jnp.float32) - S1 * s

    def rbf16(v):
        vi = lax.bitcast_convert_type(v, jnp.int32)
        return vi + 0x7FFF + (lax.shift_right_logical(vi, 16) & 1)

    def pack(v):
        hi = rbf16(v[:, 0:128]) & jnp.int32(-65536)
        lo = lax.shift_right_logical(rbf16(v[:, 128:256]), 16)
        return hi | lo

    a_ref[...] = pack(P)
    b_ref[...] = pack(Q)
    s1_ref[...] = S1
    s2_ref[...] = S2


def _prep(x, W_e):
    return pl.pallas_call(
        _prep_body,
        grid=(N // _BN1,),
        in_specs=[
            pl.BlockSpec((_BN1, D), lambda i: (i, 0)),
            pl.BlockSpec((CIN, D), lambda i: (0, 0)),
        ],
        out_specs=[
            pl.BlockSpec((_BN1, 128), lambda i: (i, 0)),
            pl.BlockSpec((_BN1, 128), lambda i: (i, 0)),
            pl.BlockSpec((_BN1, 1), lambda i: (i, 0)),
            pl.BlockSpec((_BN1, 1), lambda i: (i, 0)),
        ],
        out_shape=[
            jax.ShapeDtypeStruct((N, 128), jnp.int32),
            jax.ShapeDtypeStruct((N, 128), jnp.int32),
            jax.ShapeDtypeStruct((N, 1), jnp.float32),
            jax.ShapeDtypeStruct((N, 1), jnp.float32),
        ],
    )(x, W_e)


_KB2 = 200
_EPW = E // NW
_EPWP = _EPW + L


@functools.partial(
    pl.kernel,
    out_type=(
        jax.ShapeDtypeStruct((NW, 1, _EPW), jnp.float32),
        jax.ShapeDtypeStruct((NW, 1, _EPW), jnp.float32),
    ),
    mesh=_mesh,
    scratch_types=[
        pltpu.VMEM((_EPWP,), jnp.int32),
        pltpu.VMEM((_EPWP,), jnp.int32),
        pltpu.VMEM((N,), jnp.float32),
        pltpu.VMEM((N,), jnp.float32),
        pltpu.VMEM((_EPWP,), jnp.float32),
        pltpu.VMEM((_EPWP,), jnp.float32),
    ],
    compiler_params=pltpu.CompilerParams(needs_layout_passes=False),
)
def _stats_k(src_hbm, dst_hbm, s1_hbm, s2_hbm, s1p_hbm, s2p_hbm,
             idxs, idxd, s1b, s2b, s1pb, s2pb):
    wid = lax.axis_index("s") * NC + lax.axis_index("c")
    base0 = wid * _EPW
    pltpu.sync_copy(src_hbm.at[pl.ds(base0, _EPW)], idxs.at[pl.ds(0, _EPW)])
    pltpu.sync_copy(dst_hbm.at[pl.ds(base0, _EPW)], idxd.at[pl.ds(0, _EPW)])
    pltpu.sync_copy(s1_hbm, s1b)
    pltpu.sync_copy(s2_hbm, s2b)

    zero16 = jnp.zeros((L,), jnp.int32)
    idxs[pl.ds(_EPW, L)] = zero16
    idxd[pl.ds(_EPW, L)] = zero16

    def stats(i, c):
        sv = idxs[pl.ds(i * L, L)]
        dv = idxd[pl.ds(i * L, L)]
        s1pb[pl.ds(i * L, L)] = (plsc.load_gather(s1b, [sv])
                                 + plsc.load_gather(s1b, [dv]))
        s2pb[pl.ds(i * L, L)] = (plsc.load_gather(s2b, [sv])
                                 + plsc.load_gather(s2b, [dv]))
        return c

    lax.fori_loop(0, (_EPW + L - 1) // L, stats, 0)
    pltpu.sync_copy(s1pb.at[pl.ds(0, _EPW)], s1p_hbm.at[wid, 0])
    pltpu.sync_copy(s2pb.at[pl.ds(0, _EPW)], s2p_hbm.at[wid, 0])


@functools.partial(
    pl.kernel,
    out_type=(
        jax.ShapeDtypeStruct((E, 128), jnp.int32),
        jax.ShapeDtypeStruct((E, 128), jnp.int32),
    ),
    mesh=_mesh,
    scratch_types=[
        pltpu.VMEM((_EPW,), jnp.int32),
        pltpu.VMEM((_EPW,), jnp.int32),
        pltpu.VMEM((_KB2, 128), jnp.int32),
        pltpu.VMEM((_KB2, 128), jnp.int32),
        pltpu.SemaphoreType.DMA,
        pltpu.SemaphoreType.DMA,
        pltpu.SemaphoreType.DMA,
        pltpu.SemaphoreType.DMA,
    ],
    compiler_params=pltpu.CompilerParams(needs_layout_passes=False),
)
def _gather_k(src_hbm, dst_hbm, atab_hbm, btab_hbm,
              ga_hbm, gb_hbm,
              idxs, idxd, bufa, bufb,
              gsa, gsb, wsa, wsb):
    wid = lax.axis_index("s") * NC + lax.axis_index("c")
    base0 = wid * _EPW
    pltpu.sync_copy(src_hbm.at[pl.ds(base0, _EPW)], idxs)
    pltpu.sync_copy(dst_hbm.at[pl.ds(base0, _EPW)], idxd)

    def blk(b, carry):
        base = b * _KB2
        cga = pltpu.async_copy(atab_hbm.at[idxs.at[pl.ds(base, _KB2)]],
                               bufa, gsa)
        cgb = pltpu.async_copy(btab_hbm.at[idxd.at[pl.ds(base, _KB2)]],
                               bufb, gsb)
        cga.wait()
        cwa = pltpu.async_copy(bufa, ga_hbm.at[pl.ds(base0 + base, _KB2)],
                               wsa)
        cgb.wait()
        cwb = pltpu.async_copy(bufb, gb_hbm.at[pl.ds(base0 + base, _KB2)],
                               wsb)
        cwa.wait()
        cwb.wait()
        return carry

    lax.fori_loop(0, _EPW // _KB2, blk, 0)


_BE3 = E // NW


def _edge_body(ga_ref, gb_ref, ea_ref, s1p_ref, s2p_ref, we_ref, be_ref,
               wa_ref, h_ref, attn_ref, af_ref):
    ea = ea_ref[...]
    t = jnp.sum(ea, axis=1, keepdims=True)
    u = jnp.sum(ea * ea, axis=1, keepdims=True)
    s1p = s1p_ref[0, 0, :][:, None]
    s2p = s2p_ref[0, 0, :][:, None]
    mu = (s1p + t) / 528.0
    var = (s2p + u) / 528.0 - mu * mu
    rsig = lax.rsqrt(var + EPS)
    W = we_ref[...]
    s = jnp.sum(W, axis=0)[None, :] / 528.0
    gaw = ga_ref[...]
    gbw = gb_ref[...]
    mhi = jnp.int32(-65536)

    def unpack(w, k):
        bits = w & mhi if k == 0 else lax.shift_left(w, 16)
        return lax.bitcast_convert_type(bits, jnp.float32)

    attn = None
    for k in range(2):
        cs = slice(128 * k, 128 * (k + 1))
        gf = unpack(gaw, k) + unpack(gbw, k)
        R = jnp.dot(ea, W[256:272, cs], preferred_element_type=jnp.float32)
        hp = (gf + R - t * s[:, cs]) * rsig + be_ref[...][None, cs]
        h = jnp.where(hp > 0, hp, 0.01 * hp)
        h_ref[:, cs] = h
        pa = jnp.dot(h, wa_ref[cs, :], preferred_element_type=jnp.float32)
        attn = pa if attn is None else attn + pa
    attn_ref[...] = attn
    af_ref[0, 0, :] = attn[:, 0]


def _edge(ga, gb, edge_attr, s1p, s2p, W_e, b_e, W_a):
    return pl.pallas_call(
        _edge_body,
        grid=(E // _BE3,),
        in_specs=[
            pl.BlockSpec((_BE3, 128), lambda i: (i, 0)),
            pl.BlockSpec((_BE3, 128), lambda i: (i, 0)),
            pl.BlockSpec((_BE3, DE), lambda i: (i, 0)),
            pl.BlockSpec((1, 1, _BE3), lambda i: (i, 0, 0)),
            pl.BlockSpec((1, 1, _BE3), lambda i: (i, 0, 0)),
            pl.BlockSpec((CIN, D), lambda i: (0, 0)),
            pl.BlockSpec((D,), lambda i: (0,)),
            pl.BlockSpec((D, 1), lambda i: (0, 0)),
        ],
        out_specs=[
            pl.BlockSpec((_BE3, D), lambda i: (i, 0)),
            pl.BlockSpec((_BE3, 1), lambda i: (i, 0)),
            pl.BlockSpec((1, 1, _BE3), lambda i: (i, 0, 0)),
        ],
        out_shape=[
            jax.ShapeDtypeStruct((E, D), jnp.float32),
            jax.ShapeDtypeStruct((E, 1), jnp.float32),
            jax.ShapeDtypeStruct((NW, 1, _BE3), jnp.float32),
        ],
    )(ga, gb, edge_attr, s1p, s2p, W_e, b_e, W_a)


@functools.partial(
    pl.kernel,
    out_type=jax.ShapeDtypeStruct((NW, N), jnp.float32),
    mesh=_mesh,
    scratch_types=[
        pltpu.VMEM((_EPW,), jnp.int32),
        pltpu.VMEM((_EPW,), jnp.float32),
        pltpu.VMEM((N,), jnp.float32),
    ],
    compiler_params=pltpu.CompilerParams(needs_layout_passes=False),
)
def _segmax_k(dst_hbm, a_hbm, mparts_hbm, dbuf, abuf, mbuf):
    wid = lax.axis_index("s") * NC + lax.axis_index("c")
    base = wid * _EPW
    pltpu.sync_copy(dst_hbm.at[pl.ds(base, _EPW)], dbuf)
    pltpu.sync_copy(a_hbm.at[pl.ds(base, _EPW)], abuf)

    def initb(i, c):
        mbuf[pl.ds(i * L, L)] = jnp.full((L,), NEG, jnp.float32)
        return c

    lax.fori_loop(0, N // L, initb, 0)

    def ebody(i, c):
        dv = dbuf[pl.ds(i * L, L)]
        av = abuf[pl.ds(i * L, L)]

        for _ in range(L):
            cur = plsc.load_gather(mbuf, [dv])
            plsc.store_scatter(mbuf, [dv], av, mask=av > cur)
        return c

    lax.fori_loop(0, _EPW // L, ebody, 0)
    pltpu.sync_copy(mbuf, mparts_hbm.at[wid])


def _mmerge_body(mp_ref, m_ref):
    m_ref[...] = jnp.max(mp_ref[...], axis=0, keepdims=True)


def _mmerge(mparts):
    return pl.pallas_call(
        _mmerge_body,
        grid=(1,),
        in_specs=[pl.BlockSpec((NW, N), lambda i: (0, 0))],
        out_specs=pl.BlockSpec((1, N), lambda i: (0, 0)),
        out_shape=jax.ShapeDtypeStruct((1, N), jnp.float32),
    )(mparts)


_KB6 = 80
_EPC = E // NS
_NB6 = _EPC // _KB6
_NPT = 10240
_RPT = _NPT // NS
_CH = D // NC
_DR = _NPT // 128


@functools.partial(
    pl.kernel,
    out_type=(
        jax.ShapeDtypeStruct((_NPT, D), jnp.float32),
        jax.ShapeDtypeStruct((_DR, 128), jnp.float32),
    ),
    mesh=_mesh,
    scratch_types=[
        pltpu.VMEM((N,), jnp.float32),
        pltpu.VMEM((_DR, 128), jnp.float32),
        pltpu.VMEM((_KB6,), jnp.int32),
        pltpu.VMEM((_KB6,), jnp.int32),
        pltpu.VMEM((_KB6,), jnp.float32),
        pltpu.VMEM((_KB6,), jnp.float32),
        pltpu.VMEM((_KB6,), jnp.float32),
        pltpu.VMEM((_KB6, _CH), jnp.float32),
        pltpu.VMEM((_KB6, _CH), jnp.float32),
        pltpu.VMEM((8, 128), jnp.float32),
        pltpu.VMEM_SHARED((_NPT, _CH), jnp.float32),
        pltpu.SemaphoreType.DMA,
        pltpu.SemaphoreType.DMA,
        pltpu.SemaphoreType.DMA,
        pltpu.SemaphoreType.DMA,
        pltpu.SemaphoreType.DMA,
        pltpu.SemaphoreType.DMA,
    ],
    compiler_params=pltpu.CompilerParams(needs_layout_passes=False),
)
def _soft_k(dst_hbm, a_hbm, h_hbm, m_hbm, numer_hbm, den_hbm,
            mbuf, dpbuf, dbuf0, dbuf1, abuf0, abuf1, exb, hbuf0, hbuf1,
            dob, accum, hsem0, hsem1, ssem0, ssem1, isem0, isem1):
    c = lax.axis_index("c")
    sid = lax.axis_index("s")
    colbase = c * _CH
    db = (dbuf0, dbuf1)
    ab = (abuf0, abuf1)
    hb = (hbuf0, hbuf1)
    hsem = (hsem0, hsem1)
    ssem = (ssem0, ssem1)
    isem = (isem0, isem1)
    pltpu.sync_copy(m_hbm.at[0], mbuf)

    zero16 = jnp.zeros((L,), jnp.float32)

    def zdp(e, cc):
        for j in range(8):
            dpbuf[e, pl.ds(j * L, L)] = zero16
        return cc

    lax.fori_loop(0, _DR, zdp, 0)

    def zh(e, cc):
        for j in range(_CH // L):
            hbuf0[e, pl.ds(j * L, L)] = zero16
        return cc

    lax.fori_loop(0, _KB6, zh, 0)
    r0 = sid * _RPT
    for zi in range(_RPT // _KB6):
        pltpu.sync_copy(hbuf0.at[pl.ds(0, _KB6)],
                        accum.at[pl.ds(r0 + zi * _KB6, _KB6)])
    plsc.subcore_barrier()

    def _hslice(b):
        base = sid * _EPC + b * _KB6
        return h_hbm.at[pl.ds(base, _KB6), pl.ds(colbase, _CH)]

    def idxa(b, p):
        base = sid * _EPC + b * _KB6
        pltpu.async_copy(dst_hbm.at[pl.ds(base, _KB6)], db[p], isem[p])
        pltpu.async_copy(a_hbm.at[pl.ds(base, _KB6)], ab[p], isem[p])

    def wait_idxa(b, p):
        base = sid * _EPC + b * _KB6
        pltpu.make_async_copy(dst_hbm.at[pl.ds(base, _KB6)], db[p],
                              isem[p]).wait()
        pltpu.make_async_copy(a_hbm.at[pl.ds(base, _KB6)], ab[p],
                              isem[p]).wait()

    def chunkloop(p):
        def chunk(i, cc):
            dv = db[p][pl.ds(i * L, L)]
            mv = plsc.load_gather(mbuf, [dv])
            av = ab[p][pl.ds(i * L, L)]
            ex = jnp.exp(av - mv)
            exb[pl.ds(i * L, L)] = ex

            @pl.when(c == 0)
            def _():
                row = lax.shift_right_logical(dv, 7)
                col = lax.bitwise_and(dv, jnp.full((L,), 127, jnp.int32))
                plsc.addupdate_scatter(dpbuf, [row, col], ex)

            return cc

        lax.fori_loop(0, _KB6 // L, chunk, 0)

    def scaleloop(p):
        def scale(i, cc):
            exv = exb[pl.ds(i * L, L)]
            for l in range(L):
                e = i * L + l
                exs = exv[l]
                for j in range(_CH // L):
                    hb[p][e, pl.ds(j * L, L)] = (
                        hb[p][e, pl.ds(j * L, L)] * exs)
            return cc

        lax.fori_loop(0, _KB6 // L, scale, 0)

    def slot(b, p, wait_other, nxt):
        wait_idxa(b, p)
        chunkloop(p)
        if wait_other:
            pltpu.make_async_copy(hb[1 - p], accum.at[db[1 - p]],
                                  ssem[1 - p]).wait()
        if nxt:
            idxa(b + 1, 1 - p)
            pltpu.async_copy(_hslice(b + 1), hb[1 - p], hsem[1 - p])
        pltpu.make_async_copy(_hslice(b), hb[p], hsem[p]).wait()
        scaleloop(p)
        pltpu.async_copy(hb[p], accum.at[db[p]], ssem[p], add=True)

    idxa(0, 0)
    pltpu.async_copy(_hslice(0), hb[0], hsem[0])
    slot(0, 0, False, True)

    def pipepair(b2, carry):
        b = 2 * b2 + 1
        slot(b, 1, True, True)
        slot(b + 1, 0, True, True)
        return carry

    lax.fori_loop(0, (_NB6 - 3) // 2, pipepair, 0)
    slot(_NB6 - 2, 1, True, True)
    slot(_NB6 - 1, 0, True, False)
    pltpu.make_async_copy(hb[0], accum.at[db[0]], ssem[0]).wait()
    plsc.subcore_barrier()
    pltpu.sync_copy(accum.at[pl.ds(r0, _RPT)],
                    numer_hbm.at[pl.ds(r0, _RPT), pl.ds(colbase, _CH)])
    plsc.subcore_barrier()

    @pl.when(c == 0)
    def _():
        pltpu.sync_copy(dpbuf, accum.at[pl.ds(sid * _DR, _DR)])

    plsc.subcore_barrier()

    @pl.when((c == 0) & (sid < _DR // 8))
    def _():
        for half in range(2):
            for tt in range(8):
                t = half * 8 + tt
                pltpu.sync_copy(accum.at[pl.ds(t * _DR + sid * 8, 8)],
                                hbuf0.at[pl.ds(tt * 8, 8)])

            def dmerge(i, cc):
                r = i // 8
                j = i % 8
                if half == 0:
                    acc = jnp.zeros((L,), jnp.float32)
                else:
                    acc = dob[r, pl.ds(j * L, L)]
                for tt in range(8):
                    acc = acc + hbuf0[tt * 8 + r, pl.ds(j * L, L)]
                dob[r, pl.ds(j * L, L)] = acc
                return cc

            lax.fori_loop(0, 64, dmerge, 0)
        pltpu.sync_copy(dob, den_hbm.at[pl.ds(sid * 8, 8)])


_BN7 = 1000


def _final_body(x_ref, nu_ref, den_ref, wn_ref, bn_ref, z_ref):
    den = den_ref[...]
    af = nu_ref[...] / jnp.maximum(den, 1e-30)
    feat = jnp.concatenate([x_ref[...], af], axis=1)
    mu = jnp.mean(feat, axis=1, keepdims=True)
    var = jnp.mean((feat - mu) ** 2, axis=1, keepdims=True)
    fn = (feat - mu) * lax.rsqrt(var + EPS)
    z = jnp.dot(fn, wn_ref[...], preferred_element_type=jnp.float32)
    z_ref[...] = jnp.maximum(z + bn_ref[...][None, :], 0.0)


def _final(x, numer, den, W_n, b_n):
    return pl.pallas_call(
        _final_body,
        grid=(N // _BN7,),
        in_specs=[
            pl.BlockSpec((_BN7, D), lambda i: (i, 0)),
            pl.BlockSpec((_BN7, D), lambda i: (i, 0)),
            pl.BlockSpec((_BN7, 1), lambda i: (i, 0)),
            pl.BlockSpec((2 * D, D), lambda i: (0, 0)),
            pl.BlockSpec((D,), lambda i: (0,)),
        ],
        out_specs=pl.BlockSpec((_BN7, D), lambda i: (i, 0)),
        out_shape=jax.ShapeDtypeStruct((N, D), jnp.float32),
    )(x, numer, den, W_n, b_n)


def kernel(x, edge_index, edge_attr, W_e, b_e, W_a, W_n, b_n):
    ei = edge_index.astype(jnp.int32)
    src = ei[0]
    dst = ei[1]
    atab, btab, s1c, s2c = _prep(x, W_e)
    s1p, s2p = _stats_k(src, dst, s1c[:, 0], s2c[:, 0])
    ga, gb = _gather_k(src, dst, atab, btab)
    h, attn, af3 = _edge(ga, gb, edge_attr, s1p, s2p, W_e, b_e, W_a)
    a_flat = af3.reshape(E)
    mparts = _segmax_k(dst, a_flat)
    mmerged = _mmerge(mparts)
    numer, den_pad = _soft_k(dst, a_flat, h, mmerged)
    den = den_pad.reshape(_NPT)[:N].reshape(N, 1)
    z = _final(x, numer, den, W_n, b_n)
    return z, h, attn
